# Initial kernel scaffold; baseline (speedup 1.0000x reference)
import jax
import jax.numpy as jnp
from jax.experimental import pallas as pl


def kernel(n_feat, e_feat, edge_index, Wi, bi, Wh, bh, Wo, bo, att_W, att_b, W1, b1, W2, b2):
    raise NotImplementedError("write your pallas kernel here")


# speedup vs baseline: 1.6303x; 1.6303x over previous
import functools

import jax
import jax.numpy as jnp
from jax import lax
from jax.experimental import pallas as pl
from jax.experimental.pallas import tpu as pltpu
from jax.experimental.pallas import tpu_sc as plsc

H = 128
NSUB = 16
NCORE = 2
K = 128
WT = K * NSUB
MSTEPS = 7


def _mm_kernel(x_ref, w_ref, b_ref, o_ref, *, relu):
    acc = jnp.dot(x_ref[...], w_ref[...], preferred_element_type=jnp.float32)
    acc = acc + b_ref[...]
    o_ref[...] = jnp.maximum(acc, 0.0) if relu else acc


def _mm(x, w, b, *, relu, block_rows):
    r, kd = x.shape
    _, hd = w.shape
    assert r % block_rows == 0
    return pl.pallas_call(
        functools.partial(_mm_kernel, relu=relu),
        grid=(r // block_rows,),
        in_specs=[
            pl.BlockSpec((block_rows, kd), lambda i: (i, 0)),
            pl.BlockSpec((kd, hd), lambda i: (0, 0)),
            pl.BlockSpec((1, hd), lambda i: (0, 0)),
        ],
        out_specs=pl.BlockSpec((block_rows, hd), lambda i: (i, 0)),
        out_shape=jax.ShapeDtypeStruct((r, hd), jnp.float32),
    )(x, w, b.reshape(1, hd))


def _mm2_kernel(x1_ref, w1_ref, x2_ref, w2_ref, b_ref, o_ref, *, relu):
    acc = jnp.dot(x1_ref[...], w1_ref[...], preferred_element_type=jnp.float32)
    acc = acc + jnp.dot(x2_ref[...], w2_ref[...],
                        preferred_element_type=jnp.float32)
    acc = acc + b_ref[...]
    o_ref[...] = jnp.maximum(acc, 0.0) if relu else acc


def _mm2(x1, w1, x2, w2, b, *, relu, block_rows):
    r, k1 = x1.shape
    _, k2 = x2.shape
    _, hd = w1.shape
    assert r % block_rows == 0
    return pl.pallas_call(
        functools.partial(_mm2_kernel, relu=relu),
        grid=(r // block_rows,),
        in_specs=[
            pl.BlockSpec((block_rows, k1), lambda i: (i, 0)),
            pl.BlockSpec((k1, hd), lambda i: (0, 0)),
            pl.BlockSpec((block_rows, k2), lambda i: (i, 0)),
            pl.BlockSpec((k2, hd), lambda i: (0, 0)),
            pl.BlockSpec((1, hd), lambda i: (0, 0)),
        ],
        out_specs=pl.BlockSpec((block_rows, hd), lambda i: (i, 0)),
        out_shape=jax.ShapeDtypeStruct((r, hd), jnp.float32),
    )(x1, w1, x2, w2, b.reshape(1, hd))


def _scores_kernel(hv_ref, a_ref, ab_ref, sc_ref, mx_ref, *, blk, n_real):
    i = pl.program_id(0)
    s = jnp.dot(hv_ref[...], a_ref[...], preferred_element_type=jnp.float32)
    s = s + ab_ref[...]
    rid = lax.broadcasted_iota(jnp.int32, s.shape, 0) + i * blk
    s = jnp.where(rid < n_real, s, -1e30)
    sc_ref[...] = s
    m = jnp.max(s, axis=0, keepdims=True)

    @pl.when(i == 0)
    def _():
        mx_ref[...] = jnp.full((8, 16), -jnp.inf, jnp.float32)

    mx_ref[0:1, :] = jnp.maximum(mx_ref[0:1, :], m)


def _pool_kernel(sc_ref, hv_ref, mx_ref, gw_ref, ws_ref):
    i = pl.program_id(0)

    @pl.when(i == 0)
    def _():
        gw_ref[...] = jnp.zeros((16, H), jnp.float32)
        ws_ref[...] = jnp.zeros((8, 16), jnp.float32)

    w = jnp.exp(sc_ref[...] - mx_ref[0:1, :])
    ws_ref[0:1, :] = ws_ref[0:1, :] + jnp.sum(w, axis=0, keepdims=True)
    gw_ref[...] = gw_ref[...] + lax.dot_general(
        w, hv_ref[...], (((0,), (0,)), ((), ())),
        preferred_element_type=jnp.float32)


def _head_kernel(gw_ref, ws_ref, w1_ref, b1_ref, w2_ref, b2_ref, o_ref):
    inv = 1.0 / ws_ref[0:1, :]
    rows = []
    for t in range(12):
        gt = jnp.maximum(gw_ref[t:t + 1, :] * inv[0, t], 0.0)
        o1 = jnp.dot(gt, w1_ref[t], preferred_element_type=jnp.float32)
        o1 = jnp.maximum(o1 + b1_ref[t:t + 1, :], 0.0)
        lt = jnp.dot(o1, w2_ref[t], preferred_element_type=jnp.float32)
        lt = lt + b2_ref[t:t + 1, :]
        mt = jnp.max(lt, axis=1, keepdims=True)
        et = jnp.exp(lt - mt)
        pt = et / jnp.sum(et, axis=1, keepdims=True)
        rows.append(jnp.concatenate(
            [lt, pt, jnp.zeros((1, H - 16), jnp.float32)], axis=1))
    rows.append(jnp.zeros((4, H), jnp.float32))
    o_ref[...] = jnp.concatenate(rows, axis=0)


def _sc_pass(table, lin, srcp, ldstp, meta, *, n_out, ep, ch, chp, write_h0,
             lin_idx=None):
    zchunk = chp // NSUB
    nzfull = zchunk // K
    ztail = zchunk - nzfull * K
    cfull = ch - (NSUB - 1) * zchunk
    mesh = plsc.VectorSubcoreMesh(core_axis_name="c", subcore_axis_name="s")

    out_type = [jax.ShapeDtypeStruct((n_out, H), jnp.float32)]
    if write_h0:
        out_type.append(jax.ShapeDtypeStruct((ep, H), jnp.float32))
    out_type = tuple(out_type) if write_h0 else out_type[0]
    gather_lin = lin_idx is not None

    @functools.partial(
        pl.kernel,
        out_type=out_type,
        mesh=mesh,
        scratch_types=[
            pltpu.VMEM_SHARED((chp, H), jnp.float32),
            pltpu.VMEM((K,), jnp.int32),
            pltpu.VMEM((K,), jnp.int32),
            pltpu.VMEM((K,), jnp.int32),
            pltpu.VMEM((K, H), jnp.float32),
            pltpu.VMEM((K, H), jnp.float32),
            pltpu.VMEM((16,), jnp.int32),
            pltpu.SemaphoreType.DMA,
            pltpu.SemaphoreType.DMA,
        ],
    )
    def kern(table_h, lin_h, srcp_h, ldstp_h, permp_h, meta_h, *refs):
        if write_h0:
            agg_h, h0_h = refs[0], refs[1]
            refs = refs[2:]
        else:
            agg_h = refs[0]
            h0_h = None
            refs = refs[1:]
        (acc, src_v, ldst_v, perm_v, rows_v, lin_v, meta_v,
         sem, sem2) = refs
        cid = lax.axis_index("c")
        sid = lax.axis_index("s")
        pltpu.sync_copy(meta_h.at[pl.ds(cid * 16, 16)], meta_v)

        mv = meta_v[...]
        for lb in range(4):
            nw = mv[2 * lb]
            st = mv[2 * lb + 1]

            def zrow(k, c):
                for j in range(8):
                    rows_v[k, pl.ds(j * 16, 16)] = jnp.zeros(
                        (16,), jnp.float32)
                return c

            lax.fori_loop(0, K, zrow, 0)
            zbase = sid * zchunk
            for z in range(nzfull):
                pltpu.sync_copy(rows_v, acc.at[pl.ds(zbase + z * K, K)])
            if ztail:
                pltpu.sync_copy(rows_v.at[pl.ds(0, ztail)],
                                acc.at[pl.ds(zbase + nzfull * K, ztail)])
            plsc.subcore_barrier()

            def win(w, c):
                off = pl.multiple_of(st + w * WT + sid * K, K)
                pltpu.sync_copy(srcp_h.at[pl.ds(off, K)], src_v)
                pltpu.sync_copy(ldstp_h.at[pl.ds(off, K)], ldst_v)
                gdma = pltpu.async_copy(table_h.at[src_v], rows_v, sem)
                if gather_lin:
                    pltpu.sync_copy(permp_h.at[pl.ds(off, K)], perm_v)
                    pltpu.async_copy(lin_h.at[perm_v], lin_v, sem2).wait()
                else:
                    pltpu.sync_copy(lin_h.at[pl.ds(off, K)], lin_v)
                gdma.wait()

                def rowbody(k, c2):
                    for j in range(8):
                        s = pl.ds(j * 16, 16)
                        rows_v[k, s] = jnp.maximum(
                            rows_v[k, s] + lin_v[k, s], 0.0)
                    return c2

                lax.fori_loop(0, K, rowbody, 0)
                if write_h0:
                    pltpu.sync_copy(rows_v, h0_h.at[pl.ds(off, K)])
                pltpu.sync_copy(rows_v, acc.at[ldst_v], add=True)
                return c

            lax.fori_loop(0, nw, win, 0)
            plsc.subcore_barrier()

            out_base = (cid * 4 + lb) * ch + sid * zchunk
            pltpu.sync_copy(acc.at[pl.ds(sid * zchunk, cfull)],
                            agg_h.at[pl.ds(out_base, cfull)])

            @pl.when(sid < NSUB - 1)
            def _():
                pltpu.sync_copy(
                    acc.at[pl.ds(sid * zchunk + cfull, zchunk - cfull)],
                    agg_h.at[pl.ds(out_base + cfull, zchunk - cfull)])

            plsc.subcore_barrier()

    if lin_idx is None:
        lin_idx = jnp.zeros((8,), jnp.int32)
    return kern(table, lin, srcp, ldstp, lin_idx, meta)


def kernel(n_feat, e_feat, edge_index, Wi, bi, Wh, bh, Wo, bo,
           att_W, att_b, W1, b1, W2, b2):
    n = n_feat.shape[0]
    e = e_feat.shape[0]
    d_in = n_feat.shape[1]
    ch = ((n + 7) // 8 + 127) // 128 * 128
    npad = 8 * ch
    chp = ch + 128
    epg = NCORE * NSUB * K
    ep = (e + 8 * WT + epg - 1) // epg * epg

    src = edge_index[0].astype(jnp.int32)
    dst = edge_index[1].astype(jnp.int32)
    chunk = dst // ch

    counts = jnp.bincount(chunk, length=8)
    padded = (counts + WT - 1) // WT * WT
    starts = jnp.concatenate([jnp.zeros((1,), padded.dtype),
                              jnp.cumsum(padded)[:7]]).astype(jnp.int32)
    rank = jnp.zeros((e,), jnp.int32)
    for b in range(8):
        m = chunk == b
        rank = jnp.where(m, jnp.cumsum(m.astype(jnp.int32)) - 1, rank)
    pos = starts[chunk] + rank

    ar = jnp.arange(ep, dtype=jnp.int32)
    src_p = (ar % n).at[pos].set(src)
    perm_p = (ar % e).at[pos].set(ar[:e])
    ldst_p = jnp.full((ep,), ch, jnp.int32).at[pos].set(dst - chunk * ch)
    nwin = (padded // WT).astype(jnp.int32)
    meta = jnp.zeros((32,), jnp.int32)
    for b in range(8):
        base = (b // 4) * 16 + (b % 4) * 2
        meta = meta.at[base].set(nwin[b]).at[base + 1].set(starts[b])

    d_pad = (d_in + 7) // 8 * 8
    nf = jnp.pad(n_feat, ((0, npad - n), (0, d_pad - d_in)))
    wi_top = jnp.pad(Wi[:d_in], ((0, d_pad - d_in), (0, 0)))
    wi_bot = jnp.pad(Wi[d_in:], ((0, 16 - (Wi.shape[0] - d_in)), (0, 0)))
    ef16 = jnp.pad(e_feat, ((0, 0), (0, 16 - e_feat.shape[1])))
    wo_top = jnp.pad(Wo[:d_in], ((0, d_pad - d_in), (0, 0)))
    wo_bot = Wo[d_in:]
    amat = jnp.pad(att_W[:, :, 0].T, ((0, 0), (0, 4)))
    ab = jnp.pad(att_b[:, 0], (0, 4))
    w2p = jnp.pad(W2, ((0, 0), (0, 0), (0, 6)))
    b2p = jnp.pad(b2, ((0, 0), (0, 6)), constant_values=-1e30)

    p_tab = _mm(nf, wi_top, jnp.zeros((H,), jnp.float32),
                relu=False, block_rows=1024)
    q = _mm(ef16, wi_bot, bi, relu=False, block_rows=1600)

    agg, h0b = _sc_pass(p_tab, q, src_p, ldst_p, meta, n_out=npad, ep=ep,
                        ch=ch, chp=chp, write_h0=True, lin_idx=perm_p)
    for _ in range(MSTEPS):
        agg_w = _mm(agg, Wh, bh, relu=False, block_rows=1024)
        agg = _sc_pass(agg_w, h0b, src_p, ldst_p, meta,
                       n_out=npad, ep=ep, ch=ch, chp=chp, write_h0=False)

    hv = _mm2(nf, wo_top, agg, wo_bot, bo, relu=True, block_rows=1024)

    blk = 1024
    nblk = npad // blk
    scores, smax = pl.pallas_call(
        functools.partial(_scores_kernel, blk=blk, n_real=n),
        grid=(nblk,),
        in_specs=[
            pl.BlockSpec((blk, H), lambda i: (i, 0)),
            pl.BlockSpec((H, 16), lambda i: (0, 0)),
            pl.BlockSpec((1, 16), lambda i: (0, 0)),
        ],
        out_specs=[
            pl.BlockSpec((blk, 16), lambda i: (i, 0)),
            pl.BlockSpec((8, 16), lambda i: (0, 0)),
        ],
        out_shape=[
            jax.ShapeDtypeStruct((npad, 16), jnp.float32),
            jax.ShapeDtypeStruct((8, 16), jnp.float32),
        ],
    )(hv, amat, ab.reshape(1, 16))

    gw, ws = pl.pallas_call(
        _pool_kernel,
        grid=(nblk,),
        in_specs=[
            pl.BlockSpec((blk, 16), lambda i: (i, 0)),
            pl.BlockSpec((blk, H), lambda i: (i, 0)),
            pl.BlockSpec((8, 16), lambda i: (0, 0)),
        ],
        out_specs=[
            pl.BlockSpec((16, H), lambda i: (0, 0)),
            pl.BlockSpec((8, 16), lambda i: (0, 0)),
        ],
        out_shape=[
            jax.ShapeDtypeStruct((16, H), jnp.float32),
            jax.ShapeDtypeStruct((8, 16), jnp.float32),
        ],
    )(scores, hv, smax)

    head = pl.pallas_call(
        _head_kernel,
        in_specs=[
            pl.BlockSpec((16, H), lambda: (0, 0)),
            pl.BlockSpec((8, 16), lambda: (0, 0)),
            pl.BlockSpec((12, H, 64), lambda: (0, 0, 0)),
            pl.BlockSpec((12, 64), lambda: (0, 0)),
            pl.BlockSpec((12, 64, 8), lambda: (0, 0, 0)),
            pl.BlockSpec((12, 8), lambda: (0, 0)),
        ],
        out_specs=pl.BlockSpec((16, H), lambda: (0, 0)),
        out_shape=jax.ShapeDtypeStruct((16, H), jnp.float32),
    )(gw, ws, W1, b1, w2p, b2p)

    logits = head[:12, 0:2]
    preds = head[:12, 8:10]
    return (logits, preds)


# speedup vs baseline: 1.9073x; 1.1699x over previous
import functools

import jax
import jax.numpy as jnp
from jax import lax
from jax.experimental import pallas as pl
from jax.experimental.pallas import tpu as pltpu
from jax.experimental.pallas import tpu_sc as plsc

H = 128
NSUB = 16
NCORE = 2
K = 128
WT = K * NSUB
MSTEPS = 7


def _mm_kernel(x_ref, w_ref, b_ref, o_ref, *, relu):
    acc = jnp.dot(x_ref[...], w_ref[...], preferred_element_type=jnp.float32)
    acc = acc + b_ref[...]
    o_ref[...] = jnp.maximum(acc, 0.0) if relu else acc


def _mm(x, w, b, *, relu, block_rows):
    r, kd = x.shape
    _, hd = w.shape
    assert r % block_rows == 0
    return pl.pallas_call(
        functools.partial(_mm_kernel, relu=relu),
        grid=(r // block_rows,),
        in_specs=[
            pl.BlockSpec((block_rows, kd), lambda i: (i, 0)),
            pl.BlockSpec((kd, hd), lambda i: (0, 0)),
            pl.BlockSpec((1, hd), lambda i: (0, 0)),
        ],
        out_specs=pl.BlockSpec((block_rows, hd), lambda i: (i, 0)),
        out_shape=jax.ShapeDtypeStruct((r, hd), jnp.float32),
    )(x, w, b.reshape(1, hd))


def _mm2_kernel(x1_ref, w1_ref, x2_ref, w2_ref, b_ref, o_ref, *, relu):
    acc = jnp.dot(x1_ref[...], w1_ref[...], preferred_element_type=jnp.float32)
    acc = acc + jnp.dot(x2_ref[...], w2_ref[...],
                        preferred_element_type=jnp.float32)
    acc = acc + b_ref[...]
    o_ref[...] = jnp.maximum(acc, 0.0) if relu else acc


def _mm2(x1, w1, x2, w2, b, *, relu, block_rows):
    r, k1 = x1.shape
    _, k2 = x2.shape
    _, hd = w1.shape
    assert r % block_rows == 0
    return pl.pallas_call(
        functools.partial(_mm2_kernel, relu=relu),
        grid=(r // block_rows,),
        in_specs=[
            pl.BlockSpec((block_rows, k1), lambda i: (i, 0)),
            pl.BlockSpec((k1, hd), lambda i: (0, 0)),
            pl.BlockSpec((block_rows, k2), lambda i: (i, 0)),
            pl.BlockSpec((k2, hd), lambda i: (0, 0)),
            pl.BlockSpec((1, hd), lambda i: (0, 0)),
        ],
        out_specs=pl.BlockSpec((block_rows, hd), lambda i: (i, 0)),
        out_shape=jax.ShapeDtypeStruct((r, hd), jnp.float32),
    )(x1, w1, x2, w2, b.reshape(1, hd))


def _scores_kernel(hv_ref, a_ref, ab_ref, sc_ref, mx_ref, *, blk, n_real):
    i = pl.program_id(0)
    s = jnp.dot(hv_ref[...], a_ref[...], preferred_element_type=jnp.float32)
    s = s + ab_ref[...]
    rid = lax.broadcasted_iota(jnp.int32, s.shape, 0) + i * blk
    s = jnp.where(rid < n_real, s, -1e30)
    sc_ref[...] = s
    m = jnp.max(s, axis=0, keepdims=True)

    @pl.when(i == 0)
    def _():
        mx_ref[...] = jnp.full((8, 16), -jnp.inf, jnp.float32)

    mx_ref[0:1, :] = jnp.maximum(mx_ref[0:1, :], m)


def _pool_kernel(sc_ref, hv_ref, mx_ref, gw_ref, ws_ref):
    i = pl.program_id(0)

    @pl.when(i == 0)
    def _():
        gw_ref[...] = jnp.zeros((16, H), jnp.float32)
        ws_ref[...] = jnp.zeros((8, 16), jnp.float32)

    w = jnp.exp(sc_ref[...] - mx_ref[0:1, :])
    ws_ref[0:1, :] = ws_ref[0:1, :] + jnp.sum(w, axis=0, keepdims=True)
    gw_ref[...] = gw_ref[...] + lax.dot_general(
        w, hv_ref[...], (((0,), (0,)), ((), ())),
        preferred_element_type=jnp.float32)


def _head_kernel(gw_ref, ws_ref, w1_ref, b1_ref, w2_ref, b2_ref, o_ref):
    inv = 1.0 / ws_ref[0:1, :]
    rows = []
    for t in range(12):
        gt = jnp.maximum(gw_ref[t:t + 1, :] * inv[0, t], 0.0)
        o1 = jnp.dot(gt, w1_ref[t], preferred_element_type=jnp.float32)
        o1 = jnp.maximum(o1 + b1_ref[t:t + 1, :], 0.0)
        lt = jnp.dot(o1, w2_ref[t], preferred_element_type=jnp.float32)
        lt = lt + b2_ref[t:t + 1, :]
        mt = jnp.max(lt, axis=1, keepdims=True)
        et = jnp.exp(lt - mt)
        pt = et / jnp.sum(et, axis=1, keepdims=True)
        rows.append(jnp.concatenate(
            [lt, pt, jnp.zeros((1, H - 16), jnp.float32)], axis=1))
    rows.append(jnp.zeros((4, H), jnp.float32))
    o_ref[...] = jnp.concatenate(rows, axis=0)


def _sc_pass(table, lin, srcp, ldstp, meta, *, n_out, ep, ch, chp, write_h0,
             lin_idx=None):
    zchunk = chp // NSUB
    nzfull = zchunk // K
    ztail = zchunk - nzfull * K
    cfull = ch - (NSUB - 1) * zchunk
    mesh = plsc.VectorSubcoreMesh(core_axis_name="c", subcore_axis_name="s")

    out_type = [jax.ShapeDtypeStruct((n_out, H), jnp.float32)]
    if write_h0:
        out_type.append(jax.ShapeDtypeStruct((ep, H), jnp.float32))
    out_type = tuple(out_type) if write_h0 else out_type[0]
    gather_lin = lin_idx is not None

    @functools.partial(
        pl.kernel,
        out_type=out_type,
        mesh=mesh,
        scratch_types=[
            pltpu.VMEM_SHARED((chp, H), jnp.float32),
            pltpu.VMEM((K,), jnp.int32),
            pltpu.VMEM((K,), jnp.int32),
            pltpu.VMEM((K,), jnp.int32),
            pltpu.VMEM((K, H), jnp.float32),
            pltpu.VMEM((K, H), jnp.float32),
            pltpu.VMEM((K,), jnp.int32),
            pltpu.VMEM((K,), jnp.int32),
            pltpu.VMEM((K,), jnp.int32),
            pltpu.VMEM((K, H), jnp.float32),
            pltpu.VMEM((K, H), jnp.float32),
            pltpu.VMEM((16,), jnp.int32),
            pltpu.SemaphoreType.DMA,
            pltpu.SemaphoreType.DMA,
            pltpu.SemaphoreType.DMA,
            pltpu.SemaphoreType.DMA,
        ],
    )
    def kern(table_h, lin_h, srcp_h, ldstp_h, permp_h, meta_h, *refs):
        if write_h0:
            agg_h, h0_h = refs[0], refs[1]
            refs = refs[2:]
        else:
            agg_h = refs[0]
            h0_h = None
            refs = refs[1:]
        (acc, src_a, ldst_a, perm_a, rows_a, lin_a,
         src_b, ldst_b, perm_b, rows_b, lin_b, meta_v,
         gsa, lsa, gsb, lsb) = refs
        cid = lax.axis_index("c")
        sid = lax.axis_index("s")
        pltpu.sync_copy(meta_h.at[pl.ds(cid * 16, 16)], meta_v)

        mv = meta_v[...]
        for lb in range(4):
            nw = mv[2 * lb]
            st = mv[2 * lb + 1]

            def zrow(k, c):
                for j in range(8):
                    rows_a[k, pl.ds(j * 16, 16)] = jnp.zeros(
                        (16,), jnp.float32)
                return c

            lax.fori_loop(0, K, zrow, 0)
            zbase = sid * zchunk
            for z in range(nzfull):
                pltpu.sync_copy(rows_a, acc.at[pl.ds(zbase + z * K, K)])
            if ztail:
                pltpu.sync_copy(rows_a.at[pl.ds(0, ztail)],
                                acc.at[pl.ds(zbase + nzfull * K, ztail)])
            plsc.subcore_barrier()

            def woff(w):
                return pl.multiple_of(st + w * WT + sid * K, K)

            def prefetch(w, srcv, ldstv, permv, rowsv, linv, gsem, lsem):
                off = woff(w)
                pltpu.sync_copy(srcp_h.at[pl.ds(off, K)], srcv)
                pltpu.sync_copy(ldstp_h.at[pl.ds(off, K)], ldstv)
                pltpu.async_copy(table_h.at[srcv], rowsv, gsem)
                if gather_lin:
                    pltpu.sync_copy(permp_h.at[pl.ds(off, K)], permv)
                    pltpu.async_copy(lin_h.at[permv], linv, lsem)
                else:
                    pltpu.async_copy(lin_h.at[pl.ds(off, K)], linv, lsem)

            def consume(w, srcv, ldstv, permv, rowsv, linv, gsem, lsem):
                pltpu.make_async_copy(table_h.at[srcv], rowsv, gsem).wait()
                if gather_lin:
                    pltpu.make_async_copy(lin_h.at[permv], linv, lsem).wait()
                else:
                    pltpu.make_async_copy(
                        lin_h.at[pl.ds(woff(w), K)], linv, lsem).wait()

                def rowbody(k, c2):
                    for j in range(8):
                        s = pl.ds(j * 16, 16)
                        rowsv[k, s] = jnp.maximum(
                            rowsv[k, s] + linv[k, s], 0.0)
                    return c2

                lax.fori_loop(0, K, rowbody, 0)
                if write_h0:
                    pltpu.sync_copy(rowsv, h0_h.at[pl.ds(woff(w), K)])
                pltpu.sync_copy(rowsv, acc.at[ldstv], add=True)

            slot_a = (src_a, ldst_a, perm_a, rows_a, lin_a, gsa, lsa)
            slot_b = (src_b, ldst_b, perm_b, rows_b, lin_b, gsb, lsb)

            @pl.when(nw > 0)
            def _():
                prefetch(0, *slot_a)

            def pair(i, c):
                w0 = 2 * i
                w1 = w0 + 1

                @pl.when(w1 < nw)
                def _():
                    prefetch(w1, *slot_b)

                consume(w0, *slot_a)

                @pl.when(w0 + 2 < nw)
                def _():
                    prefetch(w0 + 2, *slot_a)

                @pl.when(w1 < nw)
                def _():
                    consume(w1, *slot_b)

                return c

            lax.fori_loop(0, (nw + 1) // 2, pair, 0)
            plsc.subcore_barrier()

            out_base = (cid * 4 + lb) * ch + sid * zchunk
            pltpu.sync_copy(acc.at[pl.ds(sid * zchunk, cfull)],
                            agg_h.at[pl.ds(out_base, cfull)])

            @pl.when(sid < NSUB - 1)
            def _():
                pltpu.sync_copy(
                    acc.at[pl.ds(sid * zchunk + cfull, zchunk - cfull)],
                    agg_h.at[pl.ds(out_base + cfull, zchunk - cfull)])

            plsc.subcore_barrier()

    if lin_idx is None:
        lin_idx = jnp.zeros((8,), jnp.int32)
    return kern(table, lin, srcp, ldstp, lin_idx, meta)


def kernel(n_feat, e_feat, edge_index, Wi, bi, Wh, bh, Wo, bo,
           att_W, att_b, W1, b1, W2, b2):
    n = n_feat.shape[0]
    e = e_feat.shape[0]
    d_in = n_feat.shape[1]
    ch = ((n + 7) // 8 + 127) // 128 * 128
    npad = 8 * ch
    chp = ch + 128
    epg = NCORE * NSUB * K
    ep = (e + 8 * WT + epg - 1) // epg * epg

    src = edge_index[0].astype(jnp.int32)
    dst = edge_index[1].astype(jnp.int32)
    chunk = dst // ch

    counts = jnp.bincount(chunk, length=8)
    padded = (counts + WT - 1) // WT * WT
    starts = jnp.concatenate([jnp.zeros((1,), padded.dtype),
                              jnp.cumsum(padded)[:7]]).astype(jnp.int32)
    rank = jnp.zeros((e,), jnp.int32)
    for b in range(8):
        m = chunk == b
        rank = jnp.where(m, jnp.cumsum(m.astype(jnp.int32)) - 1, rank)
    pos = starts[chunk] + rank

    ar = jnp.arange(ep, dtype=jnp.int32)
    src_p = (ar % n).at[pos].set(src)
    perm_p = (ar % e).at[pos].set(ar[:e])
    ldst_p = jnp.full((ep,), ch, jnp.int32).at[pos].set(dst - chunk * ch)
    nwin = (padded // WT).astype(jnp.int32)
    meta = jnp.zeros((32,), jnp.int32)
    for b in range(8):
        base = (b // 4) * 16 + (b % 4) * 2
        meta = meta.at[base].set(nwin[b]).at[base + 1].set(starts[b])

    d_pad = (d_in + 7) // 8 * 8
    nf = jnp.pad(n_feat, ((0, npad - n), (0, d_pad - d_in)))
    wi_top = jnp.pad(Wi[:d_in], ((0, d_pad - d_in), (0, 0)))
    wi_bot = jnp.pad(Wi[d_in:], ((0, 16 - (Wi.shape[0] - d_in)), (0, 0)))
    ef16 = jnp.pad(e_feat, ((0, 0), (0, 16 - e_feat.shape[1])))
    wo_top = jnp.pad(Wo[:d_in], ((0, d_pad - d_in), (0, 0)))
    wo_bot = Wo[d_in:]
    amat = jnp.pad(att_W[:, :, 0].T, ((0, 0), (0, 4)))
    ab = jnp.pad(att_b[:, 0], (0, 4))
    w2p = jnp.pad(W2, ((0, 0), (0, 0), (0, 6)))
    b2p = jnp.pad(b2, ((0, 0), (0, 6)), constant_values=-1e30)

    p_tab = _mm(nf, wi_top, jnp.zeros((H,), jnp.float32),
                relu=False, block_rows=1024)
    q = _mm(ef16, wi_bot, bi, relu=False, block_rows=1600)

    agg, h0b = _sc_pass(p_tab, q, src_p, ldst_p, meta, n_out=npad, ep=ep,
                        ch=ch, chp=chp, write_h0=True, lin_idx=perm_p)
    for _ in range(MSTEPS):
        agg_w = _mm(agg, Wh, bh, relu=False, block_rows=1024)
        agg = _sc_pass(agg_w, h0b, src_p, ldst_p, meta,
                       n_out=npad, ep=ep, ch=ch, chp=chp, write_h0=False)

    hv = _mm2(nf, wo_top, agg, wo_bot, bo, relu=True, block_rows=1024)

    blk = 1024
    nblk = npad // blk
    scores, smax = pl.pallas_call(
        functools.partial(_scores_kernel, blk=blk, n_real=n),
        grid=(nblk,),
        in_specs=[
            pl.BlockSpec((blk, H), lambda i: (i, 0)),
            pl.BlockSpec((H, 16), lambda i: (0, 0)),
            pl.BlockSpec((1, 16), lambda i: (0, 0)),
        ],
        out_specs=[
            pl.BlockSpec((blk, 16), lambda i: (i, 0)),
            pl.BlockSpec((8, 16), lambda i: (0, 0)),
        ],
        out_shape=[
            jax.ShapeDtypeStruct((npad, 16), jnp.float32),
            jax.ShapeDtypeStruct((8, 16), jnp.float32),
        ],
    )(hv, amat, ab.reshape(1, 16))

    gw, ws = pl.pallas_call(
        _pool_kernel,
        grid=(nblk,),
        in_specs=[
            pl.BlockSpec((blk, 16), lambda i: (i, 0)),
            pl.BlockSpec((blk, H), lambda i: (i, 0)),
            pl.BlockSpec((8, 16), lambda i: (0, 0)),
        ],
        out_specs=[
            pl.BlockSpec((16, H), lambda i: (0, 0)),
            pl.BlockSpec((8, 16), lambda i: (0, 0)),
        ],
        out_shape=[
            jax.ShapeDtypeStruct((16, H), jnp.float32),
            jax.ShapeDtypeStruct((8, 16), jnp.float32),
        ],
    )(scores, hv, smax)

    head = pl.pallas_call(
        _head_kernel,
        in_specs=[
            pl.BlockSpec((16, H), lambda: (0, 0)),
            pl.BlockSpec((8, 16), lambda: (0, 0)),
            pl.BlockSpec((12, H, 64), lambda: (0, 0, 0)),
            pl.BlockSpec((12, 64), lambda: (0, 0)),
            pl.BlockSpec((12, 64, 8), lambda: (0, 0, 0)),
            pl.BlockSpec((12, 8), lambda: (0, 0)),
        ],
        out_specs=pl.BlockSpec((16, H), lambda: (0, 0)),
        out_shape=jax.ShapeDtypeStruct((16, H), jnp.float32),
    )(gw, ws, W1, b1, w2p, b2p)

    logits = head[:12, 0:2]
    preds = head[:12, 8:10]
    return (logits, preds)
